# baseline (speedup 1.0000x reference)
import jax
import jax.numpy as jnp
from jax import lax
from jax.experimental import pallas as pl
from jax.experimental.pallas import tpu as pltpu
from jax.experimental.pallas import tpu_sc as plsc

_B, _L = 4096, 20
_BT = _B * _L
_EMBED = 128
_PROJ = 256
TRAVEL_ROWS = 1001

_NC, _NS = 2, 16
_NW = _NC * _NS
_CH = 160
_SPLITS = (30720, 25600, 25600)
_BLK = 5120


def _make_sc_gather(size):
    bpw = size // _NW
    nchunk = bpw // _CH

    def body(hotel_hbm, travel_hbm, hid_hbm, tid_hbm,
             out_h_hbm, out_t_hbm,
             idx_h, idx_t, trv_spmem, hbuf0, hbuf1, tbuf0, tbuf1,
             gsh0, gsh1, gst0, gst1, wsh0, wsh1, wst0, wst1):
        hbufs, tbufs = (hbuf0, hbuf1), (tbuf0, tbuf1)
        gsems_h, gsems_t = (gsh0, gsh1), (gst0, gst1)
        wsems_h, wsems_t = (wsh0, wsh1), (wst0, wst1)
        wid = lax.axis_index("s") * _NC + lax.axis_index("c")
        base = wid * bpw
        @pl.when(lax.axis_index("s") == 0)
        def _():
            pltpu.sync_copy(travel_hbm, trv_spmem)
        plsc.subcore_barrier()
        pltpu.sync_copy(hid_hbm.at[pl.ds(base, bpw)], idx_h)
        pltpu.sync_copy(tid_hbm.at[pl.ds(base, bpw)], idx_t)

        def g_h(c):
            return pltpu.async_copy(
                hotel_hbm.at[idx_h.at[pl.ds(c * _CH, _CH)]],
                hbufs[c % 2], gsems_h[c % 2])

        def g_t(c):
            return pltpu.async_copy(
                trv_spmem.at[idx_t.at[pl.ds(c * _CH, _CH)]],
                tbufs[c % 2], gsems_t[c % 2])

        def w_h(c):
            return pltpu.async_copy(
                hbufs[c % 2], out_h_hbm.at[pl.ds(base + c * _CH, _CH)],
                wsems_h[c % 2])

        def w_t(c):
            return pltpu.async_copy(
                tbufs[c % 2], out_t_hbm.at[pl.ds(base + c * _CH, _CH)],
                wsems_t[c % 2])

        gh = {0: g_h(0)}
        gt = {0: g_t(0)}
        wh, wt = {}, {}
        for c in range(nchunk):
            if c >= 1:
                wh[c - 1].wait()
                wt[c - 1].wait()
            if c + 1 < nchunk:
                gh[c + 1] = g_h(c + 1)
                gt[c + 1] = g_t(c + 1)
            gh[c].wait()
            wh[c] = w_h(c)
            gt[c].wait()
            wt[c] = w_t(c)
        wh[nchunk - 1].wait()
        wt[nchunk - 1].wait()

    return pl.kernel(
        body,
        out_type=(
            jax.ShapeDtypeStruct((size, _EMBED), jnp.float32),
            jax.ShapeDtypeStruct((size, _EMBED), jnp.float32),
        ),
        mesh=plsc.VectorSubcoreMesh(core_axis_name="c",
                                    subcore_axis_name="s"),
        scratch_types=[
            pltpu.VMEM((bpw,), jnp.int32),
            pltpu.VMEM((bpw,), jnp.int32),
            pltpu.VMEM_SHARED((TRAVEL_ROWS, _EMBED), jnp.float32),
            pltpu.VMEM((_CH, _EMBED), jnp.float32),
            pltpu.VMEM((_CH, _EMBED), jnp.float32),
            pltpu.VMEM((_CH, _EMBED), jnp.float32),
            pltpu.VMEM((_CH, _EMBED), jnp.float32),
        ] + [pltpu.SemaphoreType.DMA] * 8,
    )


_SC_GATHERS = {}


def _sc_gather(size, *args):
    if size not in _SC_GATHERS:
        _SC_GATHERS[size] = _make_sc_gather(size)
    return _SC_GATHERS[size](*args)


def _tc_body(hot, trv, gcol, dcol, gtab, dtab,
             wh, bh, wt, bt, wg, bg, wd, bd, out):
    f32 = jnp.float32
    h = jnp.tanh(jnp.dot(hot[...], wh[...], preferred_element_type=f32)
                 + bh[...])
    t = jnp.tanh(jnp.dot(trv[...], wt[...], preferred_element_type=f32)
                 + bt[...])
    gp = jnp.tanh(jnp.dot(gtab[...], wg[...], preferred_element_type=f32)
                  + bg[...])
    dp = jnp.tanh(jnp.dot(dtab[...], wd[...], preferred_element_type=f32)
                  + bd[...])
    g0 = gp[0:1, :]
    dg = gp[1:2, :] - g0
    d0 = dp[0:1, :]
    dd = dp[1:2, :] - d0
    cmat = jnp.concatenate([g0 * d0, dg * d0, g0 * dd, dg * dd], axis=0)
    p = h * t
    qt = lax.dot_general(cmat, p, (((1,), (1,)), ((), ())),
                         preferred_element_type=f32)
    gf = gcol[0]
    df = dcol[0]
    out[0] = (qt[0:1, :] + gf * qt[1:2, :] + df * qt[2:3, :]
              + (gf * df) * qt[3:4, :])


def _tc_compute(hot, trv, gflat, dflat, gtab, dtab,
                wh, bh, wt, bt, wg, bg, wd, bd):
    nblk = hot.shape[0] // _BLK
    grid = (nblk,)
    row_spec = pl.BlockSpec((_BLK, _EMBED), lambda i: (i, 0))
    lane_spec = pl.BlockSpec((1, 1, _BLK), lambda i: (i, 0, 0))
    tab_spec = pl.BlockSpec((2, _EMBED), lambda i: (0, 0))
    w_spec = pl.BlockSpec((_EMBED, _PROJ), lambda i: (0, 0))
    b_spec = pl.BlockSpec((1, _PROJ), lambda i: (0, 0))
    return pl.pallas_call(
        _tc_body,
        grid=grid,
        in_specs=[row_spec, row_spec, lane_spec, lane_spec,
                  tab_spec, tab_spec,
                  w_spec, b_spec, w_spec, b_spec,
                  w_spec, b_spec, w_spec, b_spec],
        out_specs=lane_spec,
        out_shape=jax.ShapeDtypeStruct((nblk, 1, _BLK), jnp.float32),
    )(hot, trv, gflat, dflat, gtab, dtab,
      wh, bh, wt, bt, wg, bg, wd, bd)


def kernel(hotel_id, travel_purpose, gender, desktop,
           hotel_table, travel_table, gender_table, device_table,
           W_h, b_h, W_t, b_t, W_g, b_g, W_d, b_d):
    nblk = _BT // _BLK
    offs = [0]
    for s in _SPLITS:
        offs.append(offs[-1] + s)
    hid, tid = [], []
    for i, size in enumerate(_SPLITS):
        r0, r1 = offs[i] // _L, offs[i + 1] // _L
        hid.append(hotel_id[r0:r1].reshape(size).astype(jnp.int32))
        tid.append(travel_purpose[r0:r1].reshape(size).astype(jnp.int32))
    gflat = gender.reshape(nblk, 1, _BLK).astype(jnp.float32)
    dflat = desktop.reshape(nblk, 1, _BLK).astype(jnp.float32)
    gathered = [
        _sc_gather(size, hotel_table, travel_table, hid[i], tid[i])
        for i, size in enumerate(_SPLITS)
    ]
    bh, bt = b_h.reshape(1, _PROJ), b_t.reshape(1, _PROJ)
    bg, bd = b_g.reshape(1, _PROJ), b_d.reshape(1, _PROJ)
    outs = []
    for i, size in enumerate(_SPLITS):
        b0, b1 = offs[i] // _BLK, offs[i + 1] // _BLK
        outs.append(
            _tc_compute(gathered[i][0], gathered[i][1],
                        gflat[b0:b1], dflat[b0:b1],
                        gender_table, device_table,
                        W_h, bh, W_t, bt, W_g, bg, W_d, bd))
    return jnp.concatenate(outs, axis=0).reshape(_B, _L)
